# baseline (speedup 1.0000x reference)
import functools

import jax
import jax.numpy as jnp
from jax import lax
from jax.experimental import pallas as pl
from jax.experimental.pallas import tpu as pltpu
from jax.experimental.pallas import tpu_sc as plsc

_N = 50000
_E = 800000
_ND = 384
_QD = 128
_H = 64
_Z = 0.8

_BN = 2000
_GRID = _N // _BN

_NTILES = 16
_CH = 128
_EPT = 25088
_ECORE = _EPT * _NTILES
_EP = _ECORE * 2
_NCH = _EPT // _CH
_APAD = 50176
_RPT = _APAD // _NTILES
_ZB = 112


def _row_spec(d):
    return pl.BlockSpec((_BN, d), lambda i: (i, 0))


def _full_spec(s):
    return pl.BlockSpec(s, lambda i: tuple(0 for _ in s))


def _acc_spec(s):
    return pl.BlockSpec(s, lambda i: tuple(0 for _ in s))


def _ka_body(x_ref, qe_ref, wq_ref, bq_ref, wni_ref, bni_ref, wmx_ref,
             wmq_ref, bmix_ref, o_ref):
    q = jnp.maximum(qe_ref[...] @ wq_ref[...] + bq_ref[...], 0.0)
    cadd = q @ wmq_ref[...] + bmix_ref[...]
    t = jnp.maximum(x_ref[...] @ wni_ref[...] + bni_ref[...], 0.0)
    o_ref[...] = jnp.maximum(t @ wmx_ref[...] + cadd, 0.0)


def _input_stage(x_, qe, p):
    wmx = p['nq_mix_W'][:_H, :]
    wmq = p['nq_mix_W'][_H:, :]
    return pl.pallas_call(
        _ka_body,
        grid=(_GRID,),
        in_specs=[
            _row_spec(_ND), _full_spec((1, _QD)), _full_spec((_QD, _H)),
            _full_spec((1, _H)), _full_spec((_ND, _H)), _full_spec((1, _H)),
            _full_spec((_H, _H)), _full_spec((_H, _H)), _full_spec((1, _H)),
        ],
        out_specs=_row_spec(_H),
        out_shape=jax.ShapeDtypeStruct((_N, _H), jnp.float32),
    )(x_, qe, p['question_input_W'], p['question_input_b'].reshape(1, _H),
      p['node_input_W'], p['node_input_b'].reshape(1, _H), wmx, wmq,
      p['nq_mix_b'].reshape(1, _H))


def _kb_body(x_ref, g_ref, c_ref, m_ref, t1w_ref, t1b_ref, t0w_ref, t0b_ref,
             gw_ref, asv_ref, adv_ref,
             h0_ref, h1_ref, h2_ref, h3_ref, h4_ref, h5_ref, h6_ref, h7_ref,
             hf_ref, as_ref, ad_ref, ms_ref, md_ref):
    i = pl.program_id(0)
    xn = x_ref[...] * g_ref[...] + c_ref[...]
    x1 = jnp.maximum(xn @ t1w_ref[...] + t1b_ref[...], 0.0)
    x0 = jnp.maximum(xn @ t0w_ref[...] + t0b_ref[...], 0.0)
    m = m_ref[...]
    xm = m * (_Z * x1 + (1.0 - _Z) * x0) + (1.0 - m) * (_Z * x0 + (1.0 - _Z) * x1)
    h = xm @ gw_ref[...]
    a_s = h @ asv_ref[...]
    a_d = h @ adv_ref[...]
    hrefs = (h0_ref, h1_ref, h2_ref, h3_ref, h4_ref, h5_ref, h6_ref, h7_ref)
    for e in range(8):
        hrefs[e][...] = h[:, e * 8:(e + 1) * 8]
    hf_ref[...] = h
    as_ref[...] = a_s
    ad_ref[...] = a_d

    @pl.when(i == 0)
    def _():
        ms_ref[...] = jnp.full((1, 1), -jnp.inf, jnp.float32)
        md_ref[...] = jnp.full((1, 1), -jnp.inf, jnp.float32)

    ms_ref[...] = jnp.maximum(ms_ref[...], jnp.max(a_s))
    md_ref[...] = jnp.maximum(md_ref[...], jnp.max(a_d))


def _pre_gat(x_raw, g, c, sg, lp):
    return pl.pallas_call(
        _kb_body,
        grid=(_GRID,),
        in_specs=[
            _row_spec(_H), _full_spec((1, _H)), _full_spec((1, _H)),
            _row_spec(1), _full_spec((_H, _H)), _full_spec((1, _H)),
            _full_spec((_H, _H)), _full_spec((1, _H)), _full_spec((_H, _H)),
            _full_spec((_H, 1)), _full_spec((_H, 1)),
        ],
        out_specs=[_row_spec(8)] * 8 + [
            _row_spec(_H), _row_spec(1), _row_spec(1),
            _acc_spec((1, 1)), _acc_spec((1, 1)),
        ],
        out_shape=[jax.ShapeDtypeStruct((_N, 8), jnp.float32)] * 8 + [
            jax.ShapeDtypeStruct((_N, _H), jnp.float32),
            jax.ShapeDtypeStruct((_N, 1), jnp.float32),
            jax.ShapeDtypeStruct((_N, 1), jnp.float32),
            jax.ShapeDtypeStruct((1, 1), jnp.float32),
            jax.ShapeDtypeStruct((1, 1), jnp.float32),
        ],
    )(x_raw, g, c, sg, lp['trans1_W'], lp['trans1_b'].reshape(1, _H),
      lp['trans0_W'], lp['trans0_b'].reshape(1, _H), lp['gat_W'],
      lp['gat_att_src'].reshape(_H, 1), lp['gat_att_dst'].reshape(_H, 1))


def _kc1_body(a0_ref, a1_ref, d0_ref, d1_ref, h_ref, as_ref, ad_ref,
              cs_ref, gb_ref, o_ref, sum_ref, sq_ref):
    i = pl.program_id(0)
    es = as_ref[...] + ad_ref[...]
    es = jnp.where(es >= 0, es, 0.2 * es) - cs_ref[...]
    exs = jnp.exp(es)
    acc = a0_ref[...] + a1_ref[...] + exs * h_ref[...]
    den = d0_ref[...] + d1_ref[...]
    o = acc / (den + exs + 1e-16) + gb_ref[...]
    o_ref[...] = o

    @pl.when(i == 0)
    def _():
        sum_ref[...] = jnp.zeros((1, _H), jnp.float32)
        sq_ref[...] = jnp.zeros((1, _H), jnp.float32)

    sum_ref[...] = sum_ref[...] + jnp.sum(o, axis=0, keepdims=True)
    sq_ref[...] = sq_ref[...] + jnp.sum(o * o, axis=0, keepdims=True)


def _gat_finalize(accs, dens, hfull, a_s, a_d, cs, gbias):
    return pl.pallas_call(
        _kc1_body,
        grid=(_GRID,),
        in_specs=[
            _row_spec(_H), _row_spec(_H), _row_spec(1), _row_spec(1),
            _row_spec(_H), _row_spec(1), _row_spec(1), _full_spec((1, 1)),
            _full_spec((1, _H)),
        ],
        out_specs=[_row_spec(_H), _acc_spec((1, _H)), _acc_spec((1, _H))],
        out_shape=[
            jax.ShapeDtypeStruct((_N, _H), jnp.float32),
            jax.ShapeDtypeStruct((1, _H), jnp.float32),
            jax.ShapeDtypeStruct((1, _H), jnp.float32),
        ],
    )(*accs, *dens, hfull, a_s, a_d, cs, gbias)


def _kc2_body(o_ref, x_ref, g1_ref, c1_ref, g_ref, c_ref, m_ref, w1_ref,
              b1_ref, w0_ref, b0_ref, y_ref, sum_ref, sq_ref):
    i = pl.program_id(0)
    xg = o_ref[...] * g1_ref[...] + c1_ref[...]
    xn = x_ref[...] * g_ref[...] + c_ref[...]
    w1 = w1_ref[...]
    w0 = w0_ref[...]
    y1 = xg @ w1[:_H, :] + xn @ w1[_H:, :] + b1_ref[...]
    y0 = xg @ w0[:_H, :] + xn @ w0[_H:, :] + b0_ref[...]
    m = m_ref[...]
    y = m * (_Z * y1 + (1.0 - _Z) * y0) + (1.0 - m) * (_Z * y0 + (1.0 - _Z) * y1)
    y_ref[...] = y

    @pl.when(i == 0)
    def _():
        sum_ref[...] = jnp.zeros((1, _H), jnp.float32)
        sq_ref[...] = jnp.zeros((1, _H), jnp.float32)

    sum_ref[...] = sum_ref[...] + jnp.sum(y, axis=0, keepdims=True)
    sq_ref[...] = sq_ref[...] + jnp.sum(y * y, axis=0, keepdims=True)


def _comb_stage(o_raw, x_raw, g1, c1, g, c, sg, lp):
    return pl.pallas_call(
        _kc2_body,
        grid=(_GRID,),
        in_specs=[
            _row_spec(_H), _row_spec(_H), _full_spec((1, _H)),
            _full_spec((1, _H)), _full_spec((1, _H)), _full_spec((1, _H)),
            _row_spec(1), _full_spec((2 * _H, _H)), _full_spec((1, _H)),
            _full_spec((2 * _H, _H)), _full_spec((1, _H)),
        ],
        out_specs=[_row_spec(_H), _acc_spec((1, _H)), _acc_spec((1, _H))],
        out_shape=[
            jax.ShapeDtypeStruct((_N, _H), jnp.float32),
            jax.ShapeDtypeStruct((1, _H), jnp.float32),
            jax.ShapeDtypeStruct((1, _H), jnp.float32),
        ],
    )(o_raw, x_raw, g1, c1, g, c, sg, lp['comb1_W'],
      lp['comb1_b'].reshape(1, _H), lp['comb0_W'], lp['comb0_b'].reshape(1, _H))


def _kh1_body(y_ref, g_ref, c_ref, am_ref, ab_ref, pw1_ref, pb1_ref, pw2_ref,
              pb2_ref, vw1_ref, vb1_ref, vw2_ref, vb2_ref,
              x_ref, lg_ref, ml_ref, sv_ref, sm_ref):
    i = pl.program_id(0)
    x = y_ref[...] * g_ref[...] + c_ref[...]
    x_ref[...] = x
    h1 = jnp.maximum(x @ pw1_ref[...] + pb1_ref[...], 0.0)
    lg = h1 @ pw2_ref[...] + pb2_ref[...]
    am = am_ref[...]
    lgm = jnp.where(am > 0.5, lg, -1000000000.0) + jnp.log(ab_ref[...] + 1e-10)
    lg_ref[...] = lgm
    v1 = jnp.maximum(x @ vw1_ref[...] + vb1_ref[...], 0.0)
    v = v1 @ vw2_ref[...] + vb2_ref[...]

    @pl.when(i == 0)
    def _():
        ml_ref[...] = jnp.full((1, 1), -jnp.inf, jnp.float32)
        sv_ref[...] = jnp.zeros((1, 1), jnp.float32)
        sm_ref[...] = jnp.zeros((1, 1), jnp.float32)

    ml_ref[...] = jnp.maximum(ml_ref[...], jnp.max(lgm))
    sv_ref[...] = sv_ref[...] + jnp.sum(v * am)
    sm_ref[...] = sm_ref[...] + jnp.sum(am)


def _head1(y_raw, g, c, am, ab, p):
    return pl.pallas_call(
        _kh1_body,
        grid=(_GRID,),
        in_specs=[
            _row_spec(_H), _full_spec((1, _H)), _full_spec((1, _H)),
            _row_spec(1), _row_spec(1), _full_spec((_H, _H)),
            _full_spec((1, _H)), _full_spec((_H, 1)), _full_spec((1, 1)),
            _full_spec((_H, _H)), _full_spec((1, _H)), _full_spec((_H, 1)),
            _full_spec((1, 1)),
        ],
        out_specs=[
            _row_spec(_H), _row_spec(1), _acc_spec((1, 1)), _acc_spec((1, 1)),
            _acc_spec((1, 1)),
        ],
        out_shape=[
            jax.ShapeDtypeStruct((_N, _H), jnp.float32),
            jax.ShapeDtypeStruct((_N, 1), jnp.float32),
            jax.ShapeDtypeStruct((1, 1), jnp.float32),
            jax.ShapeDtypeStruct((1, 1), jnp.float32),
            jax.ShapeDtypeStruct((1, 1), jnp.float32),
        ],
    )(y_raw, g, c, am, ab, p['policy_W1'], p['policy_b1'].reshape(1, _H),
      p['policy_W2'], p['policy_b2'].reshape(1, 1), p['value_W1'],
      p['value_b1'].reshape(1, _H), p['value_W2'], p['value_b2'].reshape(1, 1))


def _kh2_body(lg_ref, m_ref, ex_ref, s_ref):
    i = pl.program_id(0)
    ex = jnp.exp(lg_ref[...] - m_ref[...])
    ex_ref[...] = ex

    @pl.when(i == 0)
    def _():
        s_ref[...] = jnp.zeros((1, 1), jnp.float32)

    s_ref[...] = s_ref[...] + jnp.sum(ex)


def _head2(lg, m):
    return pl.pallas_call(
        _kh2_body,
        grid=(_GRID,),
        in_specs=[_row_spec(1), _full_spec((1, 1))],
        out_specs=[_row_spec(1), _acc_spec((1, 1))],
        out_shape=[
            jax.ShapeDtypeStruct((_N, 1), jnp.float32),
            jax.ShapeDtypeStruct((1, 1), jnp.float32),
        ],
    )(lg, m)


def _kh3_body(ex_ref, s_ref, p_ref, ent_ref):
    i = pl.program_id(0)
    p = ex_ref[...] / s_ref[...]
    p_ref[...] = p

    @pl.when(i == 0)
    def _():
        ent_ref[...] = jnp.zeros((1, 1), jnp.float32)

    ent_ref[...] = ent_ref[...] + jnp.sum(p * jnp.log(p + 1e-10))


def _head3(ex, s):
    return pl.pallas_call(
        _kh3_body,
        grid=(_GRID,),
        in_specs=[_row_spec(1), _full_spec((1, 1))],
        out_specs=[_row_spec(1), _acc_spec((1, 1))],
        out_shape=[
            jax.ShapeDtypeStruct((_N, 1), jnp.float32),
            jax.ShapeDtypeStruct((1, 1), jnp.float32),
        ],
    )(ex, s)


def _sc_edge_pass(src, dst, a_src, a_dst, hq, cvec):
    mesh = plsc.VectorSubcoreMesh(core_axis_name="c", subcore_axis_name="s")

    @functools.partial(
        pl.kernel,
        out_type=tuple(
            jax.ShapeDtypeStruct((2, _APAD, 8), jnp.float32)
            for _ in range(9)),
        mesh=mesh,
        compiler_params=pltpu.CompilerParams(needs_layout_passes=False,
                                             use_tc_tiling_on_sc=False),
        scratch_types=[
            pltpu.VMEM((_N,), jnp.float32),
            pltpu.VMEM((_N,), jnp.float32),
            pltpu.VMEM((4, _CH), jnp.int32),
            pltpu.VMEM((4, _CH), jnp.int32),
            pltpu.VMEM((4, _CH), jnp.int32),
            pltpu.VMEM((4, _CH, 8), jnp.float32),
            pltpu.VMEM((16,), jnp.float32),
            pltpu.VMEM_SHARED((_APAD, 8), jnp.float32),
            [pltpu.SemaphoreType.DMA] * 4,
            [pltpu.SemaphoreType.DMA] * 4,
            [pltpu.SemaphoreType.DMA] * 4,
        ],
    )
    def k(src_hbm, dst_hbm, asrc_hbm, adst_hbm, h0_hbm, h1_hbm, h2_hbm,
          h3_hbm, h4_hbm, h5_hbm, h6_hbm, h7_hbm, cvec_hbm, zacc_hbm,
          a0_hbm, a1_hbm, a2_hbm, a3_hbm, a4_hbm, a5_hbm, a6_hbm, a7_hbm,
          den_hbm,
          asrc_v, adst_v, sidx_v, didx_v, dloc_v, hrows_v, cvec_v,
          acc_sh, gsems, ssems, isems):
        c = lax.axis_index("c")
        s = lax.axis_index("s")
        pltpu.sync_copy(asrc_hbm, asrc_v)
        pltpu.sync_copy(adst_hbm, adst_v)
        pltpu.sync_copy(cvec_hbm, cvec_v)

        cv = cvec_v[...]
        iota = lax.iota(jnp.int32, 16)
        base = c * _ECORE + s * _EPT

        def _wait_gather(h_hbm, b):
            pltpu.make_async_copy(h_hbm.at[sidx_v.at[b]], hrows_v.at[b],
                                  gsems[b]).wait()

        def _wait_scatter(b):
            pltpu.make_async_copy(hrows_v.at[b], acc_sh.at[dloc_v.at[b]],
                                  ssems[b]).wait()

        def _wait_idx(b):
            pltpu.make_async_copy(src_hbm.at[pl.ds(0, _CH)], sidx_v.at[b],
                                  isems[b]).wait()
            pltpu.make_async_copy(dst_hbm.at[pl.ds(0, _CH)], didx_v.at[b],
                                  isems[b]).wait()

        def _ex_group(b, off, g):
            si = sidx_v[b, pl.ds(g * 16, 16)]
            di = didx_v[b, pl.ds(g * 16, 16)]
            e = (plsc.load_gather(asrc_v, [si])
                 + plsc.load_gather(adst_v, [di]))
            e = jnp.where(e >= 0, e, 0.2 * e) - cv
            ex = jnp.exp(e)
            eid = off + g * 16 + iota
            ok = eid < _E
            ex = jnp.where(ok, ex, 0.0)
            dl = jnp.where(ok, di, eid & 0x3FFF)
            dloc_v[b, pl.ds(g * 16, 16)] = dl
            return ex

        def subpass(h_hbm, accq_hbm, is_den):
            pltpu.sync_copy(zacc_hbm, acc_sh.at[pl.ds(s * _RPT, _RPT)])
            plsc.subcore_barrier()

            def _compute(b, cc):
                off = base + cc * _CH

                def grp(g, carry):
                    ex = _ex_group(b, off, g)
                    ridx = g * 16 + iota
                    if is_den:
                        lz = jnp.zeros((16,), jnp.int32)
                        plsc.store_scatter(hrows_v.at[b], [ridx, lz], ex)
                    else:
                        for l in range(8):
                            lidx = jnp.full((16,), l, jnp.int32)
                            v = plsc.load_gather(hrows_v.at[b], [ridx, lidx])
                            plsc.store_scatter(hrows_v.at[b], [ridx, lidx],
                                               v * ex)
                    return carry

                lax.fori_loop(0, _CH // 16, grp, 0)

            for b in range(4):
                pltpu.sync_copy(src_hbm.at[pl.ds(base + b * _CH, _CH)],
                                sidx_v.at[b])
                pltpu.sync_copy(dst_hbm.at[pl.ds(base + b * _CH, _CH)],
                                didx_v.at[b])
            if not is_den:
                pltpu.async_copy(h_hbm.at[sidx_v.at[0]], hrows_v.at[0],
                                 gsems[0])

            def quad(it, carry):
                for b in range(4):
                    b1 = (b + 1) % 4
                    cc = 4 * it + b
                    if not is_den:
                        _wait_gather(h_hbm, b)
                    _compute(b, cc)

                    @pl.when(cc + 4 < _NCH)
                    def _():
                        off2 = base + (cc + 4) * _CH
                        pltpu.async_copy(src_hbm.at[pl.ds(off2, _CH)],
                                         sidx_v.at[b], isems[b])
                        pltpu.async_copy(dst_hbm.at[pl.ds(off2, _CH)],
                                         didx_v.at[b], isems[b])

                    @pl.when(cc + 1 < _NCH)
                    def _():
                        @pl.when(cc >= 3)
                        def _():
                            _wait_idx(b1)
                            _wait_scatter(b1)

                        if not is_den:
                            pltpu.async_copy(h_hbm.at[sidx_v.at[b1]],
                                             hrows_v.at[b1], gsems[b1])

                    pltpu.async_copy(hrows_v.at[b], acc_sh.at[dloc_v.at[b]],
                                     ssems[b], add=True)
                return carry

            lax.fori_loop(0, _NCH // 4, quad, 0)
            for b in range(4):
                _wait_scatter(b)
            plsc.subcore_barrier()

            pltpu.sync_copy(
                acc_sh.at[pl.ds(s * _RPT, _RPT)],
                accq_hbm.at[c, pl.ds(s * _RPT, _RPT)])

        hs = (h0_hbm, h1_hbm, h2_hbm, h3_hbm, h4_hbm, h5_hbm, h6_hbm, h7_hbm)
        accqs = (a0_hbm, a1_hbm, a2_hbm, a3_hbm, a4_hbm, a5_hbm, a6_hbm,
                 a7_hbm)
        for q in range(8):
            subpass(hs[q], accqs[q], False)

        for b in range(4):
            pltpu.sync_copy(zacc_hbm.at[pl.ds(0, _CH)], hrows_v.at[b])
        subpass(h0_hbm, den_hbm, True)

    return k(src, dst, a_src, a_dst, *hq, cvec,
             jnp.zeros((_RPT, 8), jnp.float32))


def _gn_affine(ssum, ssq, weight, bias, mean_scale, eps=1e-5):
    mean = ssum / _N
    msq = ssq / _N
    var = msq - mean_scale * (2.0 - mean_scale) * mean * mean
    rinv = weight / jnp.sqrt(var + eps)
    g = rinv
    c = bias - rinv * mean_scale * mean
    return g.reshape(1, _H), c.reshape(1, _H)


def kernel(x_, edge_index, question_embeddings, subgraph_mask, action_mask,
           action_bias, params):
    sg = subgraph_mask.astype(jnp.float32).reshape(_N, 1)
    am = action_mask.astype(jnp.float32).reshape(_N, 1)
    ab = action_bias.reshape(_N, 1)
    pad = jnp.zeros((_EP - _E,), jnp.int32)
    src = jnp.concatenate([edge_index[0], pad])
    dst = jnp.concatenate([edge_index[1], pad])

    x = _input_stage(x_, question_embeddings, params)

    ones = jnp.ones((1, _H), jnp.float32)
    zeros = jnp.zeros((1, _H), jnp.float32)

    def layer_body(carry, lp):
        x, g_in, c_in = carry
        outs = _pre_gat(x, g_in, c_in, sg, lp)
        hs = tuple(outs[0:8])
        hfull, a_s, a_d, mx_s, mx_d = outs[8:13]
        csum = mx_s[0, 0] + mx_d[0, 0]
        cshift = jnp.where(csum >= 0, csum, 0.2 * csum)
        cvec = jnp.full((16,), cshift, jnp.float32)
        accs = _sc_edge_pass(src, dst, a_s.reshape(_N), a_d.reshape(_N),
                             hs, cvec)
        accq = tuple(
            jnp.concatenate([accs[e][p, :_N] for e in range(8)], axis=1)
            for p in range(2))
        dens = tuple(accs[8][p, :_N, 0:1] for p in range(2))
        o_raw, s1, q1 = _gat_finalize(
            accq, dens, hfull, a_s, a_d, cshift.reshape(1, 1),
            lp['gat_bias'].reshape(1, _H))
        g1, c1 = _gn_affine(s1[0], q1[0], lp['gn_weight'], lp['gn_bias'],
                            lp['gn_mean_scale'])
        y_raw, s2, q2 = _comb_stage(o_raw, x, g1, c1, g_in, c_in, sg, lp)
        g_out, c_out = _gn_affine(s2[0], q2[0], lp['outer_gn_weight'],
                                  lp['outer_gn_bias'],
                                  lp['outer_gn_mean_scale'])
        return (y_raw, g_out, c_out), None

    stacked = jax.tree.map(lambda *xs: jnp.stack(xs), *params['layers'])
    (x, g_in, c_in), _ = lax.scan(layer_body, (x, ones, zeros), stacked)

    xout, lg, ml, sv, sm = _head1(x, g_in, c_in, am, ab, params)
    ex, ssum = _head2(lg, ml)
    probs, ent = _head3(ex, ssum)
    entropy = -ent[0, 0]
    state_value = sv[0, 0] / jnp.maximum(sm[0, 0], 1.0)
    return probs.reshape(_N), state_value, xout, entropy


# speedup vs baseline: 1.3840x; 1.3840x over previous
import functools

import jax
import jax.numpy as jnp
from jax import lax
from jax.experimental import pallas as pl
from jax.experimental.pallas import tpu as pltpu
from jax.experimental.pallas import tpu_sc as plsc

_N = 50000
_E = 800000
_ND = 384
_QD = 128
_H = 64
_Z = 0.8

_BN = 2000
_GRID = _N // _BN

_NTILES = 16
_CH = 128
_EPT = 25088
_ECORE = _EPT * _NTILES
_EP = _ECORE * 2
_NCH = _EPT // _CH
_APAD = 50176
_RPT = _APAD // _NTILES
_ZB = 112


def _row_spec(d):
    return pl.BlockSpec((_BN, d), lambda i: (i, 0))


def _full_spec(s):
    return pl.BlockSpec(s, lambda i: tuple(0 for _ in s))


def _acc_spec(s):
    return pl.BlockSpec(s, lambda i: tuple(0 for _ in s))


def _ka_body(x_ref, qe_ref, wq_ref, bq_ref, wni_ref, bni_ref, wmx_ref,
             wmq_ref, bmix_ref, o_ref):
    q = jnp.maximum(qe_ref[...] @ wq_ref[...] + bq_ref[...], 0.0)
    cadd = q @ wmq_ref[...] + bmix_ref[...]
    t = jnp.maximum(x_ref[...] @ wni_ref[...] + bni_ref[...], 0.0)
    o_ref[...] = jnp.maximum(t @ wmx_ref[...] + cadd, 0.0)


def _input_stage(x_, qe, p):
    wmx = p['nq_mix_W'][:_H, :]
    wmq = p['nq_mix_W'][_H:, :]
    return pl.pallas_call(
        _ka_body,
        grid=(_GRID,),
        in_specs=[
            _row_spec(_ND), _full_spec((1, _QD)), _full_spec((_QD, _H)),
            _full_spec((1, _H)), _full_spec((_ND, _H)), _full_spec((1, _H)),
            _full_spec((_H, _H)), _full_spec((_H, _H)), _full_spec((1, _H)),
        ],
        out_specs=_row_spec(_H),
        out_shape=jax.ShapeDtypeStruct((_N, _H), jnp.float32),
    )(x_, qe, p['question_input_W'], p['question_input_b'].reshape(1, _H),
      p['node_input_W'], p['node_input_b'].reshape(1, _H), wmx, wmq,
      p['nq_mix_b'].reshape(1, _H))


def _kb_body(x_ref, g_ref, c_ref, m_ref, t1w_ref, t1b_ref, t0w_ref, t0b_ref,
             gw_ref, asv_ref, adv_ref,
             h0_ref, h1_ref, h2_ref, h3_ref, h4_ref, h5_ref, h6_ref, h7_ref,
             hf_ref, as_ref, ad_ref, ms_ref, md_ref):
    i = pl.program_id(0)
    xn = x_ref[...] * g_ref[...] + c_ref[...]
    x1 = jnp.maximum(xn @ t1w_ref[...] + t1b_ref[...], 0.0)
    x0 = jnp.maximum(xn @ t0w_ref[...] + t0b_ref[...], 0.0)
    m = m_ref[...]
    xm = m * (_Z * x1 + (1.0 - _Z) * x0) + (1.0 - m) * (_Z * x0 + (1.0 - _Z) * x1)
    h = xm @ gw_ref[...]
    a_s = h @ asv_ref[...]
    a_d = h @ adv_ref[...]
    hrefs = (h0_ref, h1_ref, h2_ref, h3_ref, h4_ref, h5_ref, h6_ref, h7_ref)
    for e in range(8):
        hrefs[e][...] = h[:, e * 8:(e + 1) * 8]
    hf_ref[...] = h
    as_ref[...] = a_s
    ad_ref[...] = a_d

    @pl.when(i == 0)
    def _():
        ms_ref[...] = jnp.full((1, 1), -jnp.inf, jnp.float32)
        md_ref[...] = jnp.full((1, 1), -jnp.inf, jnp.float32)

    ms_ref[...] = jnp.maximum(ms_ref[...], jnp.max(a_s))
    md_ref[...] = jnp.maximum(md_ref[...], jnp.max(a_d))


def _pre_gat(x_raw, g, c, sg, lp):
    return pl.pallas_call(
        _kb_body,
        grid=(_GRID,),
        in_specs=[
            _row_spec(_H), _full_spec((1, _H)), _full_spec((1, _H)),
            _row_spec(1), _full_spec((_H, _H)), _full_spec((1, _H)),
            _full_spec((_H, _H)), _full_spec((1, _H)), _full_spec((_H, _H)),
            _full_spec((_H, 1)), _full_spec((_H, 1)),
        ],
        out_specs=[_row_spec(8)] * 8 + [
            _row_spec(_H), _row_spec(1), _row_spec(1),
            _acc_spec((1, 1)), _acc_spec((1, 1)),
        ],
        out_shape=[jax.ShapeDtypeStruct((_N, 8), jnp.float32)] * 8 + [
            jax.ShapeDtypeStruct((_N, _H), jnp.float32),
            jax.ShapeDtypeStruct((_N, 1), jnp.float32),
            jax.ShapeDtypeStruct((_N, 1), jnp.float32),
            jax.ShapeDtypeStruct((1, 1), jnp.float32),
            jax.ShapeDtypeStruct((1, 1), jnp.float32),
        ],
    )(x_raw, g, c, sg, lp['trans1_W'], lp['trans1_b'].reshape(1, _H),
      lp['trans0_W'], lp['trans0_b'].reshape(1, _H), lp['gat_W'],
      lp['gat_att_src'].reshape(_H, 1), lp['gat_att_dst'].reshape(_H, 1))


def _kc1_body(a0_ref, a1_ref, d0_ref, d1_ref, h_ref, as_ref, ad_ref,
              cs_ref, gb_ref, o_ref, sum_ref, sq_ref):
    i = pl.program_id(0)
    es = as_ref[...] + ad_ref[...]
    es = jnp.where(es >= 0, es, 0.2 * es) - cs_ref[...]
    exs = jnp.exp(es)
    acc = a0_ref[...] + a1_ref[...] + exs * h_ref[...]
    den = d0_ref[...] + d1_ref[...]
    o = acc / (den + exs + 1e-16) + gb_ref[...]
    o_ref[...] = o

    @pl.when(i == 0)
    def _():
        sum_ref[...] = jnp.zeros((1, _H), jnp.float32)
        sq_ref[...] = jnp.zeros((1, _H), jnp.float32)

    sum_ref[...] = sum_ref[...] + jnp.sum(o, axis=0, keepdims=True)
    sq_ref[...] = sq_ref[...] + jnp.sum(o * o, axis=0, keepdims=True)


def _gat_finalize(accs, dens, hfull, a_s, a_d, cs, gbias):
    return pl.pallas_call(
        _kc1_body,
        grid=(_GRID,),
        in_specs=[
            _row_spec(_H), _row_spec(_H), _row_spec(1), _row_spec(1),
            _row_spec(_H), _row_spec(1), _row_spec(1), _full_spec((1, 1)),
            _full_spec((1, _H)),
        ],
        out_specs=[_row_spec(_H), _acc_spec((1, _H)), _acc_spec((1, _H))],
        out_shape=[
            jax.ShapeDtypeStruct((_N, _H), jnp.float32),
            jax.ShapeDtypeStruct((1, _H), jnp.float32),
            jax.ShapeDtypeStruct((1, _H), jnp.float32),
        ],
    )(*accs, *dens, hfull, a_s, a_d, cs, gbias)


def _kc2_body(o_ref, x_ref, g1_ref, c1_ref, g_ref, c_ref, m_ref, w1_ref,
              b1_ref, w0_ref, b0_ref, y_ref, sum_ref, sq_ref):
    i = pl.program_id(0)
    xg = o_ref[...] * g1_ref[...] + c1_ref[...]
    xn = x_ref[...] * g_ref[...] + c_ref[...]
    w1 = w1_ref[...]
    w0 = w0_ref[...]
    y1 = xg @ w1[:_H, :] + xn @ w1[_H:, :] + b1_ref[...]
    y0 = xg @ w0[:_H, :] + xn @ w0[_H:, :] + b0_ref[...]
    m = m_ref[...]
    y = m * (_Z * y1 + (1.0 - _Z) * y0) + (1.0 - m) * (_Z * y0 + (1.0 - _Z) * y1)
    y_ref[...] = y

    @pl.when(i == 0)
    def _():
        sum_ref[...] = jnp.zeros((1, _H), jnp.float32)
        sq_ref[...] = jnp.zeros((1, _H), jnp.float32)

    sum_ref[...] = sum_ref[...] + jnp.sum(y, axis=0, keepdims=True)
    sq_ref[...] = sq_ref[...] + jnp.sum(y * y, axis=0, keepdims=True)


def _comb_stage(o_raw, x_raw, g1, c1, g, c, sg, lp):
    return pl.pallas_call(
        _kc2_body,
        grid=(_GRID,),
        in_specs=[
            _row_spec(_H), _row_spec(_H), _full_spec((1, _H)),
            _full_spec((1, _H)), _full_spec((1, _H)), _full_spec((1, _H)),
            _row_spec(1), _full_spec((2 * _H, _H)), _full_spec((1, _H)),
            _full_spec((2 * _H, _H)), _full_spec((1, _H)),
        ],
        out_specs=[_row_spec(_H), _acc_spec((1, _H)), _acc_spec((1, _H))],
        out_shape=[
            jax.ShapeDtypeStruct((_N, _H), jnp.float32),
            jax.ShapeDtypeStruct((1, _H), jnp.float32),
            jax.ShapeDtypeStruct((1, _H), jnp.float32),
        ],
    )(o_raw, x_raw, g1, c1, g, c, sg, lp['comb1_W'],
      lp['comb1_b'].reshape(1, _H), lp['comb0_W'], lp['comb0_b'].reshape(1, _H))


def _kh1_body(y_ref, g_ref, c_ref, am_ref, ab_ref, pw1_ref, pb1_ref, pw2_ref,
              pb2_ref, vw1_ref, vb1_ref, vw2_ref, vb2_ref,
              x_ref, lg_ref, ml_ref, sv_ref, sm_ref):
    i = pl.program_id(0)
    x = y_ref[...] * g_ref[...] + c_ref[...]
    x_ref[...] = x
    h1 = jnp.maximum(x @ pw1_ref[...] + pb1_ref[...], 0.0)
    lg = h1 @ pw2_ref[...] + pb2_ref[...]
    am = am_ref[...]
    lgm = jnp.where(am > 0.5, lg, -1000000000.0) + jnp.log(ab_ref[...] + 1e-10)
    lg_ref[...] = lgm
    v1 = jnp.maximum(x @ vw1_ref[...] + vb1_ref[...], 0.0)
    v = v1 @ vw2_ref[...] + vb2_ref[...]

    @pl.when(i == 0)
    def _():
        ml_ref[...] = jnp.full((1, 1), -jnp.inf, jnp.float32)
        sv_ref[...] = jnp.zeros((1, 1), jnp.float32)
        sm_ref[...] = jnp.zeros((1, 1), jnp.float32)

    ml_ref[...] = jnp.maximum(ml_ref[...], jnp.max(lgm))
    sv_ref[...] = sv_ref[...] + jnp.sum(v * am)
    sm_ref[...] = sm_ref[...] + jnp.sum(am)


def _head1(y_raw, g, c, am, ab, p):
    return pl.pallas_call(
        _kh1_body,
        grid=(_GRID,),
        in_specs=[
            _row_spec(_H), _full_spec((1, _H)), _full_spec((1, _H)),
            _row_spec(1), _row_spec(1), _full_spec((_H, _H)),
            _full_spec((1, _H)), _full_spec((_H, 1)), _full_spec((1, 1)),
            _full_spec((_H, _H)), _full_spec((1, _H)), _full_spec((_H, 1)),
            _full_spec((1, 1)),
        ],
        out_specs=[
            _row_spec(_H), _row_spec(1), _acc_spec((1, 1)), _acc_spec((1, 1)),
            _acc_spec((1, 1)),
        ],
        out_shape=[
            jax.ShapeDtypeStruct((_N, _H), jnp.float32),
            jax.ShapeDtypeStruct((_N, 1), jnp.float32),
            jax.ShapeDtypeStruct((1, 1), jnp.float32),
            jax.ShapeDtypeStruct((1, 1), jnp.float32),
            jax.ShapeDtypeStruct((1, 1), jnp.float32),
        ],
    )(y_raw, g, c, am, ab, p['policy_W1'], p['policy_b1'].reshape(1, _H),
      p['policy_W2'], p['policy_b2'].reshape(1, 1), p['value_W1'],
      p['value_b1'].reshape(1, _H), p['value_W2'], p['value_b2'].reshape(1, 1))


def _kh2_body(lg_ref, m_ref, ex_ref, s_ref):
    i = pl.program_id(0)
    ex = jnp.exp(lg_ref[...] - m_ref[...])
    ex_ref[...] = ex

    @pl.when(i == 0)
    def _():
        s_ref[...] = jnp.zeros((1, 1), jnp.float32)

    s_ref[...] = s_ref[...] + jnp.sum(ex)


def _head2(lg, m):
    return pl.pallas_call(
        _kh2_body,
        grid=(_GRID,),
        in_specs=[_row_spec(1), _full_spec((1, 1))],
        out_specs=[_row_spec(1), _acc_spec((1, 1))],
        out_shape=[
            jax.ShapeDtypeStruct((_N, 1), jnp.float32),
            jax.ShapeDtypeStruct((1, 1), jnp.float32),
        ],
    )(lg, m)


def _kh3_body(ex_ref, s_ref, p_ref, ent_ref):
    i = pl.program_id(0)
    p = ex_ref[...] / s_ref[...]
    p_ref[...] = p

    @pl.when(i == 0)
    def _():
        ent_ref[...] = jnp.zeros((1, 1), jnp.float32)

    ent_ref[...] = ent_ref[...] + jnp.sum(p * jnp.log(p + 1e-10))


def _head3(ex, s):
    return pl.pallas_call(
        _kh3_body,
        grid=(_GRID,),
        in_specs=[_row_spec(1), _full_spec((1, 1))],
        out_specs=[_row_spec(1), _acc_spec((1, 1))],
        out_shape=[
            jax.ShapeDtypeStruct((_N, 1), jnp.float32),
            jax.ShapeDtypeStruct((1, 1), jnp.float32),
        ],
    )(ex, s)


def _sc_edge_pass(src, dst, a_src, a_dst, hq, cvec):
    mesh = plsc.VectorSubcoreMesh(core_axis_name="c", subcore_axis_name="s")

    @functools.partial(
        pl.kernel,
        out_type=tuple(
            jax.ShapeDtypeStruct((2, _APAD, 8), jnp.float32)
            for _ in range(9)),
        mesh=mesh,
        compiler_params=pltpu.CompilerParams(needs_layout_passes=False,
                                             use_tc_tiling_on_sc=False),
        scratch_types=[
            pltpu.VMEM((_N,), jnp.float32),
            pltpu.VMEM((_N,), jnp.float32),
            pltpu.VMEM((4, _CH), jnp.int32),
            pltpu.VMEM((4, _CH), jnp.int32),
            pltpu.VMEM((4, _CH), jnp.int32),
            pltpu.VMEM((4, _CH, 8), jnp.float32),
            pltpu.VMEM((16,), jnp.float32),
            pltpu.VMEM_SHARED((_APAD, 8), jnp.float32),
            [pltpu.SemaphoreType.DMA] * 4,
            [pltpu.SemaphoreType.DMA] * 4,
            [pltpu.SemaphoreType.DMA] * 4,
        ],
    )
    def k(src_hbm, dst_hbm, asrc_hbm, adst_hbm, h0_hbm, h1_hbm, h2_hbm,
          h3_hbm, h4_hbm, h5_hbm, h6_hbm, h7_hbm, cvec_hbm, zacc_hbm,
          a0_hbm, a1_hbm, a2_hbm, a3_hbm, a4_hbm, a5_hbm, a6_hbm, a7_hbm,
          den_hbm,
          asrc_v, adst_v, sidx_v, didx_v, dloc_v, hrows_v, cvec_v,
          acc_sh, gsems, ssems, isems):
        c = lax.axis_index("c")
        s = lax.axis_index("s")
        pltpu.sync_copy(asrc_hbm, asrc_v)
        pltpu.sync_copy(adst_hbm, adst_v)
        pltpu.sync_copy(cvec_hbm, cvec_v)

        cv = cvec_v[...]
        iota = lax.iota(jnp.int32, 16)
        base = c * _ECORE + s * _EPT

        def _wait_gather(h_hbm, b):
            pltpu.make_async_copy(h_hbm.at[sidx_v.at[b]], hrows_v.at[b],
                                  gsems[b]).wait()

        def _wait_scatter(b):
            pltpu.make_async_copy(hrows_v.at[b], acc_sh.at[dloc_v.at[b]],
                                  ssems[b]).wait()

        def _wait_idx(b):
            pltpu.make_async_copy(src_hbm.at[pl.ds(0, _CH)], sidx_v.at[b],
                                  isems[b]).wait()
            pltpu.make_async_copy(dst_hbm.at[pl.ds(0, _CH)], didx_v.at[b],
                                  isems[b]).wait()

        def _ex_group(b, off, g):
            si = sidx_v[b, pl.ds(g * 16, 16)]
            di = didx_v[b, pl.ds(g * 16, 16)]
            e = (plsc.load_gather(asrc_v, [si])
                 + plsc.load_gather(adst_v, [di]))
            e = jnp.where(e >= 0, e, 0.2 * e) - cv
            ex = jnp.exp(e)
            eid = off + g * 16 + iota
            ok = eid < _E
            ex = jnp.where(ok, ex, 0.0)
            dl = jnp.where(ok, di, eid & 0x3FFF)
            dloc_v[b, pl.ds(g * 16, 16)] = dl
            return ex

        def subpass(h_hbm, accq_hbm, is_den):
            pltpu.sync_copy(zacc_hbm, acc_sh.at[pl.ds(s * _RPT, _RPT)])
            plsc.subcore_barrier()

            def _compute(b, cc):
                off = base + cc * _CH

                def grp(g, carry):
                    ex = _ex_group(b, off, g)
                    ridx = g * 16 + iota
                    if is_den:
                        lz = jnp.zeros((16,), jnp.int32)
                        plsc.store_scatter(hrows_v.at[b], [ridx, lz], ex)
                    else:
                        for l in range(8):
                            lidx = jnp.full((16,), l, jnp.int32)
                            v = plsc.load_gather(hrows_v.at[b], [ridx, lidx])
                            plsc.store_scatter(hrows_v.at[b], [ridx, lidx],
                                               v * ex)
                    return carry

                lax.fori_loop(0, _CH // 16, grp, 0)

            for b in range(4):
                pltpu.sync_copy(src_hbm.at[pl.ds(base + b * _CH, _CH)],
                                sidx_v.at[b])
                pltpu.sync_copy(dst_hbm.at[pl.ds(base + b * _CH, _CH)],
                                didx_v.at[b])
            if not is_den:
                pltpu.async_copy(h_hbm.at[sidx_v.at[0]], hrows_v.at[0],
                                 gsems[0])
                pltpu.async_copy(h_hbm.at[sidx_v.at[1]], hrows_v.at[1],
                                 gsems[1])

            def quad(it, carry):
                for b in range(4):
                    b2 = (b + 2) % 4
                    cc = 4 * it + b
                    if not is_den:
                        _wait_gather(h_hbm, b)
                    _compute(b, cc)

                    @pl.when(cc + 4 < _NCH)
                    def _():
                        off2 = base + (cc + 4) * _CH
                        pltpu.async_copy(src_hbm.at[pl.ds(off2, _CH)],
                                         sidx_v.at[b], isems[b])
                        pltpu.async_copy(dst_hbm.at[pl.ds(off2, _CH)],
                                         didx_v.at[b], isems[b])

                    @pl.when(cc + 2 < _NCH)
                    def _():
                        @pl.when(cc >= 2)
                        def _():
                            _wait_idx(b2)
                            _wait_scatter(b2)

                        if not is_den:
                            pltpu.async_copy(h_hbm.at[sidx_v.at[b2]],
                                             hrows_v.at[b2], gsems[b2])

                    pltpu.async_copy(hrows_v.at[b], acc_sh.at[dloc_v.at[b]],
                                     ssems[b], add=True)
                return carry

            lax.fori_loop(0, _NCH // 4, quad, 0)
            for b in range(4):
                _wait_scatter(b)
            plsc.subcore_barrier()

            pltpu.sync_copy(
                acc_sh.at[pl.ds(s * _RPT, _RPT)],
                accq_hbm.at[c, pl.ds(s * _RPT, _RPT)])

        hs = (h0_hbm, h1_hbm, h2_hbm, h3_hbm, h4_hbm, h5_hbm, h6_hbm, h7_hbm)
        accqs = (a0_hbm, a1_hbm, a2_hbm, a3_hbm, a4_hbm, a5_hbm, a6_hbm,
                 a7_hbm)
        for q in range(8):
            subpass(hs[q], accqs[q], False)

        for b in range(4):
            pltpu.sync_copy(zacc_hbm.at[pl.ds(0, _CH)], hrows_v.at[b])
        subpass(h0_hbm, den_hbm, True)

    return k(src, dst, a_src, a_dst, *hq, cvec,
             jnp.zeros((_RPT, 8), jnp.float32))


def _gn_affine(ssum, ssq, weight, bias, mean_scale, eps=1e-5):
    mean = ssum / _N
    msq = ssq / _N
    var = msq - mean_scale * (2.0 - mean_scale) * mean * mean
    rinv = weight / jnp.sqrt(var + eps)
    g = rinv
    c = bias - rinv * mean_scale * mean
    return g.reshape(1, _H), c.reshape(1, _H)


def kernel(x_, edge_index, question_embeddings, subgraph_mask, action_mask,
           action_bias, params):
    sg = subgraph_mask.astype(jnp.float32).reshape(_N, 1)
    am = action_mask.astype(jnp.float32).reshape(_N, 1)
    ab = action_bias.reshape(_N, 1)
    pad = jnp.zeros((_EP - _E,), jnp.int32)
    src = jnp.concatenate([edge_index[0], pad])
    dst = jnp.concatenate([edge_index[1], pad])

    x = _input_stage(x_, question_embeddings, params)

    ones = jnp.ones((1, _H), jnp.float32)
    zeros = jnp.zeros((1, _H), jnp.float32)

    def layer_body(carry, lp):
        x, g_in, c_in = carry
        outs = _pre_gat(x, g_in, c_in, sg, lp)
        hs = tuple(outs[0:8])
        hfull, a_s, a_d, mx_s, mx_d = outs[8:13]
        csum = mx_s[0, 0] + mx_d[0, 0]
        cshift = jnp.where(csum >= 0, csum, 0.2 * csum)
        cvec = jnp.full((16,), cshift, jnp.float32)
        accs = _sc_edge_pass(src, dst, a_s.reshape(_N), a_d.reshape(_N),
                             hs, cvec)
        accq = tuple(
            jnp.concatenate([accs[e][p, :_N] for e in range(8)], axis=1)
            for p in range(2))
        dens = tuple(accs[8][p, :_N, 0:1] for p in range(2))
        o_raw, s1, q1 = _gat_finalize(
            accq, dens, hfull, a_s, a_d, cshift.reshape(1, 1),
            lp['gat_bias'].reshape(1, _H))
        g1, c1 = _gn_affine(s1[0], q1[0], lp['gn_weight'], lp['gn_bias'],
                            lp['gn_mean_scale'])
        y_raw, s2, q2 = _comb_stage(o_raw, x, g1, c1, g_in, c_in, sg, lp)
        g_out, c_out = _gn_affine(s2[0], q2[0], lp['outer_gn_weight'],
                                  lp['outer_gn_bias'],
                                  lp['outer_gn_mean_scale'])
        return (y_raw, g_out, c_out), None

    stacked = jax.tree.map(lambda *xs: jnp.stack(xs), *params['layers'])
    (x, g_in, c_in), _ = lax.scan(layer_body, (x, ones, zeros), stacked)

    xout, lg, ml, sv, sm = _head1(x, g_in, c_in, am, ab, params)
    ex, ssum = _head2(lg, ml)
    probs, ent = _head3(ex, ssum)
    entropy = -ent[0, 0]
    state_value = sv[0, 0] / jnp.maximum(sm[0, 0], 1.0)
    return probs.reshape(_N), state_value, xout, entropy


# speedup vs baseline: 1.4497x; 1.0475x over previous
import functools

import jax
import jax.numpy as jnp
from jax import lax
from jax.experimental import pallas as pl
from jax.experimental.pallas import tpu as pltpu
from jax.experimental.pallas import tpu_sc as plsc

_N = 50000
_E = 800000
_ND = 384
_QD = 128
_H = 64
_Z = 0.8

_BN = 2000
_GRID = _N // _BN

_NTILES = 16
_CH = 128
_EPT = 25088
_ECORE = _EPT * _NTILES
_EP = _ECORE * 2
_NCH = _EPT // _CH
_APAD = 50176
_RPT = _APAD // _NTILES
_ZB = 112


def _row_spec(d):
    return pl.BlockSpec((_BN, d), lambda i: (i, 0))


def _full_spec(s):
    return pl.BlockSpec(s, lambda i: tuple(0 for _ in s))


def _acc_spec(s):
    return pl.BlockSpec(s, lambda i: tuple(0 for _ in s))


def _ka_body(x_ref, qe_ref, wq_ref, bq_ref, wni_ref, bni_ref, wmx_ref,
             wmq_ref, bmix_ref, o_ref):
    q = jnp.maximum(qe_ref[...] @ wq_ref[...] + bq_ref[...], 0.0)
    cadd = q @ wmq_ref[...] + bmix_ref[...]
    t = jnp.maximum(x_ref[...] @ wni_ref[...] + bni_ref[...], 0.0)
    o_ref[...] = jnp.maximum(t @ wmx_ref[...] + cadd, 0.0)


def _input_stage(x_, qe, p):
    wmx = p['nq_mix_W'][:_H, :]
    wmq = p['nq_mix_W'][_H:, :]
    return pl.pallas_call(
        _ka_body,
        grid=(_GRID,),
        in_specs=[
            _row_spec(_ND), _full_spec((1, _QD)), _full_spec((_QD, _H)),
            _full_spec((1, _H)), _full_spec((_ND, _H)), _full_spec((1, _H)),
            _full_spec((_H, _H)), _full_spec((_H, _H)), _full_spec((1, _H)),
        ],
        out_specs=_row_spec(_H),
        out_shape=jax.ShapeDtypeStruct((_N, _H), jnp.float32),
    )(x_, qe, p['question_input_W'], p['question_input_b'].reshape(1, _H),
      p['node_input_W'], p['node_input_b'].reshape(1, _H), wmx, wmq,
      p['nq_mix_b'].reshape(1, _H))


def _kb_body(x_ref, g_ref, c_ref, m_ref, t1w_ref, t1b_ref, t0w_ref, t0b_ref,
             gw_ref, asv_ref, adv_ref,
             h0_ref, h1_ref, h2_ref, h3_ref, h4_ref, h5_ref, h6_ref, h7_ref,
             hf_ref, as_ref, ad_ref, ms_ref, md_ref):
    i = pl.program_id(0)
    xn = x_ref[...] * g_ref[...] + c_ref[...]
    x1 = jnp.maximum(xn @ t1w_ref[...] + t1b_ref[...], 0.0)
    x0 = jnp.maximum(xn @ t0w_ref[...] + t0b_ref[...], 0.0)
    m = m_ref[...]
    xm = m * (_Z * x1 + (1.0 - _Z) * x0) + (1.0 - m) * (_Z * x0 + (1.0 - _Z) * x1)
    h = xm @ gw_ref[...]
    a_s = h @ asv_ref[...]
    a_d = h @ adv_ref[...]
    hrefs = (h0_ref, h1_ref, h2_ref, h3_ref, h4_ref, h5_ref, h6_ref, h7_ref)
    for e in range(8):
        hrefs[e][...] = h[:, e * 8:(e + 1) * 8]
    hf_ref[...] = h
    as_ref[...] = a_s
    ad_ref[...] = a_d

    @pl.when(i == 0)
    def _():
        ms_ref[...] = jnp.full((1, 1), -jnp.inf, jnp.float32)
        md_ref[...] = jnp.full((1, 1), -jnp.inf, jnp.float32)

    ms_ref[...] = jnp.maximum(ms_ref[...], jnp.max(a_s))
    md_ref[...] = jnp.maximum(md_ref[...], jnp.max(a_d))


def _pre_gat(x_raw, g, c, sg, lp):
    return pl.pallas_call(
        _kb_body,
        grid=(_GRID,),
        in_specs=[
            _row_spec(_H), _full_spec((1, _H)), _full_spec((1, _H)),
            _row_spec(1), _full_spec((_H, _H)), _full_spec((1, _H)),
            _full_spec((_H, _H)), _full_spec((1, _H)), _full_spec((_H, _H)),
            _full_spec((_H, 1)), _full_spec((_H, 1)),
        ],
        out_specs=[_row_spec(8)] * 8 + [
            _row_spec(_H), _row_spec(1), _row_spec(1),
            _acc_spec((1, 1)), _acc_spec((1, 1)),
        ],
        out_shape=[jax.ShapeDtypeStruct((_N, 8), jnp.float32)] * 8 + [
            jax.ShapeDtypeStruct((_N, _H), jnp.float32),
            jax.ShapeDtypeStruct((_N, 1), jnp.float32),
            jax.ShapeDtypeStruct((_N, 1), jnp.float32),
            jax.ShapeDtypeStruct((1, 1), jnp.float32),
            jax.ShapeDtypeStruct((1, 1), jnp.float32),
        ],
    )(x_raw, g, c, sg, lp['trans1_W'], lp['trans1_b'].reshape(1, _H),
      lp['trans0_W'], lp['trans0_b'].reshape(1, _H), lp['gat_W'],
      lp['gat_att_src'].reshape(_H, 1), lp['gat_att_dst'].reshape(_H, 1))


def _kc1_body(a0_ref, a1_ref, d0_ref, d1_ref, h_ref, as_ref, ad_ref,
              cs_ref, gb_ref, o_ref, sum_ref, sq_ref):
    i = pl.program_id(0)
    es = as_ref[...] + ad_ref[...]
    es = jnp.where(es >= 0, es, 0.2 * es) - cs_ref[...]
    exs = jnp.exp(es)
    acc = a0_ref[...] + a1_ref[...] + exs * h_ref[...]
    den = d0_ref[...] + d1_ref[...]
    o = acc / (den + exs + 1e-16) + gb_ref[...]
    o_ref[...] = o

    @pl.when(i == 0)
    def _():
        sum_ref[...] = jnp.zeros((1, _H), jnp.float32)
        sq_ref[...] = jnp.zeros((1, _H), jnp.float32)

    sum_ref[...] = sum_ref[...] + jnp.sum(o, axis=0, keepdims=True)
    sq_ref[...] = sq_ref[...] + jnp.sum(o * o, axis=0, keepdims=True)


def _gat_finalize(accs, dens, hfull, a_s, a_d, cs, gbias):
    return pl.pallas_call(
        _kc1_body,
        grid=(_GRID,),
        in_specs=[
            _row_spec(_H), _row_spec(_H), _row_spec(1), _row_spec(1),
            _row_spec(_H), _row_spec(1), _row_spec(1), _full_spec((1, 1)),
            _full_spec((1, _H)),
        ],
        out_specs=[_row_spec(_H), _acc_spec((1, _H)), _acc_spec((1, _H))],
        out_shape=[
            jax.ShapeDtypeStruct((_N, _H), jnp.float32),
            jax.ShapeDtypeStruct((1, _H), jnp.float32),
            jax.ShapeDtypeStruct((1, _H), jnp.float32),
        ],
    )(*accs, *dens, hfull, a_s, a_d, cs, gbias)


def _kc2_body(o_ref, x_ref, g1_ref, c1_ref, g_ref, c_ref, m_ref, w1_ref,
              b1_ref, w0_ref, b0_ref, y_ref, sum_ref, sq_ref):
    i = pl.program_id(0)
    xg = o_ref[...] * g1_ref[...] + c1_ref[...]
    xn = x_ref[...] * g_ref[...] + c_ref[...]
    w1 = w1_ref[...]
    w0 = w0_ref[...]
    y1 = xg @ w1[:_H, :] + xn @ w1[_H:, :] + b1_ref[...]
    y0 = xg @ w0[:_H, :] + xn @ w0[_H:, :] + b0_ref[...]
    m = m_ref[...]
    y = m * (_Z * y1 + (1.0 - _Z) * y0) + (1.0 - m) * (_Z * y0 + (1.0 - _Z) * y1)
    y_ref[...] = y

    @pl.when(i == 0)
    def _():
        sum_ref[...] = jnp.zeros((1, _H), jnp.float32)
        sq_ref[...] = jnp.zeros((1, _H), jnp.float32)

    sum_ref[...] = sum_ref[...] + jnp.sum(y, axis=0, keepdims=True)
    sq_ref[...] = sq_ref[...] + jnp.sum(y * y, axis=0, keepdims=True)


def _comb_stage(o_raw, x_raw, g1, c1, g, c, sg, lp):
    return pl.pallas_call(
        _kc2_body,
        grid=(_GRID,),
        in_specs=[
            _row_spec(_H), _row_spec(_H), _full_spec((1, _H)),
            _full_spec((1, _H)), _full_spec((1, _H)), _full_spec((1, _H)),
            _row_spec(1), _full_spec((2 * _H, _H)), _full_spec((1, _H)),
            _full_spec((2 * _H, _H)), _full_spec((1, _H)),
        ],
        out_specs=[_row_spec(_H), _acc_spec((1, _H)), _acc_spec((1, _H))],
        out_shape=[
            jax.ShapeDtypeStruct((_N, _H), jnp.float32),
            jax.ShapeDtypeStruct((1, _H), jnp.float32),
            jax.ShapeDtypeStruct((1, _H), jnp.float32),
        ],
    )(o_raw, x_raw, g1, c1, g, c, sg, lp['comb1_W'],
      lp['comb1_b'].reshape(1, _H), lp['comb0_W'], lp['comb0_b'].reshape(1, _H))


def _kh1_body(y_ref, g_ref, c_ref, am_ref, ab_ref, pw1_ref, pb1_ref, pw2_ref,
              pb2_ref, vw1_ref, vb1_ref, vw2_ref, vb2_ref,
              x_ref, lg_ref, ml_ref, sv_ref, sm_ref):
    i = pl.program_id(0)
    x = y_ref[...] * g_ref[...] + c_ref[...]
    x_ref[...] = x
    h1 = jnp.maximum(x @ pw1_ref[...] + pb1_ref[...], 0.0)
    lg = h1 @ pw2_ref[...] + pb2_ref[...]
    am = am_ref[...]
    lgm = jnp.where(am > 0.5, lg, -1000000000.0) + jnp.log(ab_ref[...] + 1e-10)
    lg_ref[...] = lgm
    v1 = jnp.maximum(x @ vw1_ref[...] + vb1_ref[...], 0.0)
    v = v1 @ vw2_ref[...] + vb2_ref[...]

    @pl.when(i == 0)
    def _():
        ml_ref[...] = jnp.full((1, 1), -jnp.inf, jnp.float32)
        sv_ref[...] = jnp.zeros((1, 1), jnp.float32)
        sm_ref[...] = jnp.zeros((1, 1), jnp.float32)

    ml_ref[...] = jnp.maximum(ml_ref[...], jnp.max(lgm))
    sv_ref[...] = sv_ref[...] + jnp.sum(v * am)
    sm_ref[...] = sm_ref[...] + jnp.sum(am)


def _head1(y_raw, g, c, am, ab, p):
    return pl.pallas_call(
        _kh1_body,
        grid=(_GRID,),
        in_specs=[
            _row_spec(_H), _full_spec((1, _H)), _full_spec((1, _H)),
            _row_spec(1), _row_spec(1), _full_spec((_H, _H)),
            _full_spec((1, _H)), _full_spec((_H, 1)), _full_spec((1, 1)),
            _full_spec((_H, _H)), _full_spec((1, _H)), _full_spec((_H, 1)),
            _full_spec((1, 1)),
        ],
        out_specs=[
            _row_spec(_H), _row_spec(1), _acc_spec((1, 1)), _acc_spec((1, 1)),
            _acc_spec((1, 1)),
        ],
        out_shape=[
            jax.ShapeDtypeStruct((_N, _H), jnp.float32),
            jax.ShapeDtypeStruct((_N, 1), jnp.float32),
            jax.ShapeDtypeStruct((1, 1), jnp.float32),
            jax.ShapeDtypeStruct((1, 1), jnp.float32),
            jax.ShapeDtypeStruct((1, 1), jnp.float32),
        ],
    )(y_raw, g, c, am, ab, p['policy_W1'], p['policy_b1'].reshape(1, _H),
      p['policy_W2'], p['policy_b2'].reshape(1, 1), p['value_W1'],
      p['value_b1'].reshape(1, _H), p['value_W2'], p['value_b2'].reshape(1, 1))


def _kh2_body(lg_ref, m_ref, ex_ref, s_ref):
    i = pl.program_id(0)
    ex = jnp.exp(lg_ref[...] - m_ref[...])
    ex_ref[...] = ex

    @pl.when(i == 0)
    def _():
        s_ref[...] = jnp.zeros((1, 1), jnp.float32)

    s_ref[...] = s_ref[...] + jnp.sum(ex)


def _head2(lg, m):
    return pl.pallas_call(
        _kh2_body,
        grid=(_GRID,),
        in_specs=[_row_spec(1), _full_spec((1, 1))],
        out_specs=[_row_spec(1), _acc_spec((1, 1))],
        out_shape=[
            jax.ShapeDtypeStruct((_N, 1), jnp.float32),
            jax.ShapeDtypeStruct((1, 1), jnp.float32),
        ],
    )(lg, m)


def _kh3_body(ex_ref, s_ref, p_ref, ent_ref):
    i = pl.program_id(0)
    p = ex_ref[...] / s_ref[...]
    p_ref[...] = p

    @pl.when(i == 0)
    def _():
        ent_ref[...] = jnp.zeros((1, 1), jnp.float32)

    ent_ref[...] = ent_ref[...] + jnp.sum(p * jnp.log(p + 1e-10))


def _head3(ex, s):
    return pl.pallas_call(
        _kh3_body,
        grid=(_GRID,),
        in_specs=[_row_spec(1), _full_spec((1, 1))],
        out_specs=[_row_spec(1), _acc_spec((1, 1))],
        out_shape=[
            jax.ShapeDtypeStruct((_N, 1), jnp.float32),
            jax.ShapeDtypeStruct((1, 1), jnp.float32),
        ],
    )(ex, s)


def _sc_edge_pass(src, dst, a_src, a_dst, hq, cvec):
    mesh = plsc.VectorSubcoreMesh(core_axis_name="c", subcore_axis_name="s")

    @functools.partial(
        pl.kernel,
        out_type=tuple(
            jax.ShapeDtypeStruct((2, _APAD, 8), jnp.float32)
            for _ in range(9)),
        mesh=mesh,
        compiler_params=pltpu.CompilerParams(needs_layout_passes=False,
                                             use_tc_tiling_on_sc=False),
        scratch_types=[
            pltpu.VMEM((_N,), jnp.float32),
            pltpu.VMEM((_N,), jnp.float32),
            pltpu.VMEM((4, _CH), jnp.int32),
            pltpu.VMEM((4, _CH), jnp.int32),
            pltpu.VMEM((4, _CH), jnp.int32),
            pltpu.VMEM((4, _CH, 8), jnp.float32),
            pltpu.VMEM((16,), jnp.float32),
            pltpu.VMEM_SHARED((_APAD, 8), jnp.float32),
            [pltpu.SemaphoreType.DMA] * 4,
            [pltpu.SemaphoreType.DMA] * 4,
            [pltpu.SemaphoreType.DMA] * 4,
        ],
    )
    def k(src_hbm, dst_hbm, asrc_hbm, adst_hbm, h0_hbm, h1_hbm, h2_hbm,
          h3_hbm, h4_hbm, h5_hbm, h6_hbm, h7_hbm, cvec_hbm, zacc_hbm,
          a0_hbm, a1_hbm, a2_hbm, a3_hbm, a4_hbm, a5_hbm, a6_hbm, a7_hbm,
          den_hbm,
          asrc_v, adst_v, sidx_v, didx_v, dloc_v, hrows_v, cvec_v,
          acc_sh, gsems, ssems, isems):
        c = lax.axis_index("c")
        s = lax.axis_index("s")
        pltpu.sync_copy(asrc_hbm, asrc_v)
        pltpu.sync_copy(adst_hbm, adst_v)
        pltpu.sync_copy(cvec_hbm, cvec_v)

        cv = cvec_v[...]
        iota = lax.iota(jnp.int32, 16)
        base = c * _ECORE + s * _EPT

        def _wait_gather(h_hbm, b):
            pltpu.make_async_copy(h_hbm.at[sidx_v.at[b]], hrows_v.at[b],
                                  gsems[b]).wait()

        def _wait_scatter(b):
            pltpu.make_async_copy(hrows_v.at[b], acc_sh.at[dloc_v.at[b]],
                                  ssems[b]).wait()

        def _wait_idx(b):
            pltpu.make_async_copy(src_hbm.at[pl.ds(0, _CH)], sidx_v.at[b],
                                  isems[b]).wait()
            pltpu.make_async_copy(dst_hbm.at[pl.ds(0, _CH)], didx_v.at[b],
                                  isems[b]).wait()

        def _ex_group(b, off, g):
            si = sidx_v[b, pl.ds(g * 16, 16)]
            di = didx_v[b, pl.ds(g * 16, 16)]
            e = (plsc.load_gather(asrc_v, [si])
                 + plsc.load_gather(adst_v, [di]))
            e = jnp.where(e >= 0, e, 0.2 * e) - cv
            ex = jnp.exp(e)
            eid = off + g * 16 + iota
            ok = eid < _E
            ex = jnp.where(ok, ex, 0.0)
            dl = jnp.where(ok, di, eid & 0x3FFF)
            dloc_v[b, pl.ds(g * 16, 16)] = dl
            return ex

        def subpass(h_hbm, accq_hbm, is_den):
            pltpu.sync_copy(zacc_hbm, acc_sh.at[pl.ds(s * _RPT, _RPT)])
            plsc.subcore_barrier()

            def _compute(b, cc):
                off = base + cc * _CH

                def grp(g, carry):
                    ex = _ex_group(b, off, g)
                    ridx = g * 16 + iota
                    if is_den:
                        lz = jnp.zeros((16,), jnp.int32)
                        plsc.store_scatter(hrows_v.at[b], [ridx, lz], ex)
                    else:
                        for l in range(8):
                            lidx = jnp.full((16,), l, jnp.int32)
                            v = plsc.load_gather(hrows_v.at[b], [ridx, lidx])
                            plsc.store_scatter(hrows_v.at[b], [ridx, lidx],
                                               v * ex)
                    return carry

                lax.fori_loop(0, _CH // 16, grp, 0)

            for b in range(4):
                pltpu.sync_copy(src_hbm.at[pl.ds(base + b * _CH, _CH)],
                                sidx_v.at[b])
                pltpu.sync_copy(dst_hbm.at[pl.ds(base + b * _CH, _CH)],
                                didx_v.at[b])
            if not is_den:
                for b in range(3):
                    pltpu.async_copy(h_hbm.at[sidx_v.at[b]], hrows_v.at[b],
                                     gsems[b])

            def quad(it, carry):
                for b in range(4):
                    b2 = (b + 3) % 4
                    cc = 4 * it + b
                    if not is_den:
                        _wait_gather(h_hbm, b)
                    _compute(b, cc)

                    @pl.when(cc + 4 < _NCH)
                    def _():
                        off2 = base + (cc + 4) * _CH
                        pltpu.async_copy(src_hbm.at[pl.ds(off2, _CH)],
                                         sidx_v.at[b], isems[b])
                        pltpu.async_copy(dst_hbm.at[pl.ds(off2, _CH)],
                                         didx_v.at[b], isems[b])

                    @pl.when(cc + 3 < _NCH)
                    def _():
                        @pl.when(cc >= 1)
                        def _():
                            _wait_idx(b2)
                            _wait_scatter(b2)

                        if not is_den:
                            pltpu.async_copy(h_hbm.at[sidx_v.at[b2]],
                                             hrows_v.at[b2], gsems[b2])

                    pltpu.async_copy(hrows_v.at[b], acc_sh.at[dloc_v.at[b]],
                                     ssems[b], add=True)
                return carry

            lax.fori_loop(0, _NCH // 4, quad, 0)
            for b in range(4):
                _wait_scatter(b)
            plsc.subcore_barrier()

            pltpu.sync_copy(
                acc_sh.at[pl.ds(s * _RPT, _RPT)],
                accq_hbm.at[c, pl.ds(s * _RPT, _RPT)])

        hs = (h0_hbm, h1_hbm, h2_hbm, h3_hbm, h4_hbm, h5_hbm, h6_hbm, h7_hbm)
        accqs = (a0_hbm, a1_hbm, a2_hbm, a3_hbm, a4_hbm, a5_hbm, a6_hbm,
                 a7_hbm)
        for q in range(8):
            subpass(hs[q], accqs[q], False)

        for b in range(4):
            pltpu.sync_copy(zacc_hbm.at[pl.ds(0, _CH)], hrows_v.at[b])
        subpass(h0_hbm, den_hbm, True)

    return k(src, dst, a_src, a_dst, *hq, cvec,
             jnp.zeros((_RPT, 8), jnp.float32))


def _gn_affine(ssum, ssq, weight, bias, mean_scale, eps=1e-5):
    mean = ssum / _N
    msq = ssq / _N
    var = msq - mean_scale * (2.0 - mean_scale) * mean * mean
    rinv = weight / jnp.sqrt(var + eps)
    g = rinv
    c = bias - rinv * mean_scale * mean
    return g.reshape(1, _H), c.reshape(1, _H)


def kernel(x_, edge_index, question_embeddings, subgraph_mask, action_mask,
           action_bias, params):
    sg = subgraph_mask.astype(jnp.float32).reshape(_N, 1)
    am = action_mask.astype(jnp.float32).reshape(_N, 1)
    ab = action_bias.reshape(_N, 1)
    pad = jnp.zeros((_EP - _E,), jnp.int32)
    src = jnp.concatenate([edge_index[0], pad])
    dst = jnp.concatenate([edge_index[1], pad])

    x = _input_stage(x_, question_embeddings, params)

    ones = jnp.ones((1, _H), jnp.float32)
    zeros = jnp.zeros((1, _H), jnp.float32)

    def layer_body(carry, lp):
        x, g_in, c_in = carry
        outs = _pre_gat(x, g_in, c_in, sg, lp)
        hs = tuple(outs[0:8])
        hfull, a_s, a_d, mx_s, mx_d = outs[8:13]
        csum = mx_s[0, 0] + mx_d[0, 0]
        cshift = jnp.where(csum >= 0, csum, 0.2 * csum)
        cvec = jnp.full((16,), cshift, jnp.float32)
        accs = _sc_edge_pass(src, dst, a_s.reshape(_N), a_d.reshape(_N),
                             hs, cvec)
        accq = tuple(
            jnp.concatenate([accs[e][p, :_N] for e in range(8)], axis=1)
            for p in range(2))
        dens = tuple(accs[8][p, :_N, 0:1] for p in range(2))
        o_raw, s1, q1 = _gat_finalize(
            accq, dens, hfull, a_s, a_d, cshift.reshape(1, 1),
            lp['gat_bias'].reshape(1, _H))
        g1, c1 = _gn_affine(s1[0], q1[0], lp['gn_weight'], lp['gn_bias'],
                            lp['gn_mean_scale'])
        y_raw, s2, q2 = _comb_stage(o_raw, x, g1, c1, g_in, c_in, sg, lp)
        g_out, c_out = _gn_affine(s2[0], q2[0], lp['outer_gn_weight'],
                                  lp['outer_gn_bias'],
                                  lp['outer_gn_mean_scale'])
        return (y_raw, g_out, c_out), None

    stacked = jax.tree.map(lambda *xs: jnp.stack(xs), *params['layers'])
    (x, g_in, c_in), _ = lax.scan(layer_body, (x, ones, zeros), stacked)

    xout, lg, ml, sv, sm = _head1(x, g_in, c_in, am, ab, params)
    ex, ssum = _head2(lg, ml)
    probs, ent = _head3(ex, ssum)
    entropy = -ent[0, 0]
    state_value = sv[0, 0] / jnp.maximum(sm[0, 0], 1.0)
    return probs.reshape(_N), state_value, xout, entropy
